# Initial kernel scaffold; baseline (speedup 1.0000x reference)
import jax
import jax.numpy as jnp
from jax.experimental import pallas as pl


def kernel(edges, hidden, nodes, node_params, cond, ne0_W, ne0_b, ne1_W, ne1_b, g1_W, g1_b, g2_W, g2_b, g3a_W, g3a_b, g3b_W, g3b_b, ce0_W, ce0_b, ce1_W, ce1_b, fc0_W, fc0_b, fc1_W, fc1_b, fc2_W, fc2_b, fc3_W, fc3_b):
    raise NotImplementedError("write your pallas kernel here")


# speedup vs baseline: 1.4202x; 1.4202x over previous
import functools

import jax
import jax.numpy as jnp
from jax.experimental import pallas as pl

B = 32
V = 128
C = 4


def _lin(x, w_ref, b_ref):
    return jax.lax.dot_general(
        x, w_ref[...], (((1,), (1,)), ((), ())),
        preferred_element_type=jnp.float32) + b_ref[...]


def _fused_kernel(edges_ref, nodes_ref, nparams_ref, cond_ref,
                  ne0_W, ne0_b, ne1_W, ne1_b, g1_W, g1_b, g2_W, g2_b,
                  g3a_W, g3a_b, g3b_W, g3b_b, ce0_W, ce0_b, ce1_W, ce1_b,
                  fc0_W, fc0_b, fc1_W, fc1_b, fc2_W, fc2_b, fc3_W, fc3_b,
                  out_ref):
    f32 = jnp.float32
    E = edges_ref[...]
    k = jax.lax.broadcasted_iota(jnp.int32, (V * C, V), 0)
    j = jax.lax.broadcasted_iota(jnp.int32, (V * C, V), 1)
    M = jnp.where((k // C == j) & (k % C != 0), f32(1.0), f32(0.0))
    A = jnp.dot(E, M, preferred_element_type=f32)
    deg = jnp.sum(A, axis=1, keepdims=True) + 1e-8

    h = _lin(nparams_ref[...], ne0_W, ne0_b)
    h = _lin(h, ne1_W, ne1_b)
    n0 = _lin(jnp.concatenate([nodes_ref[...], h], axis=1), g1_W, g1_b)

    m1_parts = []
    for b in range(B):
        s = slice(b * V, (b + 1) * V)
        m1_parts.append(jnp.dot(A[s], n0[s], preferred_element_type=f32))
    m1 = jnp.concatenate(m1_parts, axis=0) / deg

    n1 = _lin(m1, g2_W, g2_b)

    mus = []
    for b in range(B):
        s = slice(b * V, (b + 1) * V)
        degb = deg[s]
        m2 = jnp.dot(A[s], n1[s], preferred_element_type=f32) / degb
        am2 = jnp.dot(A[s], m2, preferred_element_type=f32)
        ev = ((degb - 1e-8) * m2 - am2) / degb
        mus.append(jnp.concatenate(
            [jnp.mean(m2, axis=0, keepdims=True),
             jnp.mean(ev, axis=0, keepdims=True)], axis=1))
    mu = jnp.concatenate(mus, axis=0)

    gl = _lin(_lin(mu, g3a_W, g3a_b), g3b_W, g3b_b)
    c = _lin(_lin(cond_ref[...], ce0_W, ce0_b), ce1_W, ce1_b)
    gl = jnp.concatenate([gl, c], axis=1)
    gl = _lin(gl, fc0_W, fc0_b)
    gl = _lin(gl, fc1_W, fc1_b)
    gl = _lin(gl, fc2_W, fc2_b)
    t = gl * fc3_W[...]
    s = jnp.dot(t, jnp.ones((32, V), f32),
                preferred_element_type=f32)
    out_ref[...] = s + fc3_b[0, 0]


@functools.partial(jax.jit, static_argnames=())
def kernel(edges, hidden, nodes, node_params, cond,
           ne0_W, ne0_b, ne1_W, ne1_b, g1_W, g1_b, g2_W, g2_b,
           g3a_W, g3a_b, g3b_W, g3b_b, ce0_W, ce0_b, ce1_W, ce1_b,
           fc0_W, fc0_b, fc1_W, fc1_b, fc2_W, fc2_b, fc3_W, fc3_b):
    del hidden
    edges2d = edges.reshape(B * V, V * C)
    nodes2d = nodes.reshape(B * V, -1)
    nparams2d = node_params.reshape(B * V, -1)
    args = [edges2d, nodes2d, nparams2d, cond,
            ne0_W, ne0_b.reshape(1, -1), ne1_W, ne1_b.reshape(1, -1),
            g1_W, g1_b.reshape(1, -1), g2_W, g2_b.reshape(1, -1),
            g3a_W, g3a_b.reshape(1, -1), g3b_W, g3b_b.reshape(1, -1),
            ce0_W, ce0_b.reshape(1, -1), ce1_W, ce1_b.reshape(1, -1),
            fc0_W, fc0_b.reshape(1, -1), fc1_W, fc1_b.reshape(1, -1),
            fc2_W, fc2_b.reshape(1, -1), fc3_W, fc3_b.reshape(1, -1)]
    out = pl.pallas_call(
        _fused_kernel,
        out_shape=jax.ShapeDtypeStruct((B, V), jnp.float32),
    )(*args)
    return out[:, :1]


# speedup vs baseline: 3.2738x; 2.3051x over previous
import functools

import jax
import jax.numpy as jnp
from jax.experimental import pallas as pl

B = 32
V = 128
C = 4


def _lin(x, w_ref, b_ref):
    return jax.lax.dot_general(
        x, w_ref[...], (((1,), (1,)), ((), ())),
        preferred_element_type=jnp.float32) + b_ref[...]


def _fused_kernel(edges_ref, nodes_ref, nparams_ref, cond_ref,
                  ne0_W, ne0_b, ne1_W, ne1_b, g1_W, g1_b, g2_W, g2_b,
                  g3a_W, g3a_b, g3b_W, g3b_b, ce0_W, ce0_b, ce1_W, ce1_b,
                  fc0_W, fc0_b, fc1_W, fc1_b, fc2_W, fc2_b, fc3_W, fc3_b,
                  out_ref):
    f32 = jnp.float32
    i_idx = jax.lax.broadcasted_iota(jnp.int32, (V, V * C), 0)
    k_idx = jax.lax.broadcasted_iota(jnp.int32, (V, V * C), 1)
    Msel = jnp.where((k_idx // C == i_idx) & (k_idx % C != 0),
                     f32(1.0), f32(0.0))

    As, degs = [], []
    for b in range(B):
        Eb = edges_ref[b * V * C:(b + 1) * V * C, :]
        Ab = jnp.dot(Msel, Eb, preferred_element_type=f32)
        As.append(Ab)
        degs.append(jnp.sum(Ab, axis=1, keepdims=True) + 1e-8)

    h = _lin(nparams_ref[...], ne0_W, ne0_b)
    h = _lin(h, ne1_W, ne1_b)
    n0 = _lin(jnp.concatenate([nodes_ref[...], h], axis=1), g1_W, g1_b)

    m1_parts = []
    for b in range(B):
        s = slice(b * V, (b + 1) * V)
        m1_parts.append(
            jnp.dot(As[b], n0[s], preferred_element_type=f32) / degs[b])
    m1 = jnp.concatenate(m1_parts, axis=0)

    n1 = _lin(m1, g2_W, g2_b)

    mus = []
    for b in range(B):
        s = slice(b * V, (b + 1) * V)
        degb = degs[b]
        m2 = jnp.dot(As[b], n1[s], preferred_element_type=f32) / degb
        am2 = jnp.dot(As[b], m2, preferred_element_type=f32)
        ev = ((degb - 1e-8) * m2 - am2) / degb
        mus.append(jnp.concatenate(
            [jnp.mean(m2, axis=0, keepdims=True),
             jnp.mean(ev, axis=0, keepdims=True)], axis=1))
    mu = jnp.concatenate(mus, axis=0)

    gl = _lin(_lin(mu, g3a_W, g3a_b), g3b_W, g3b_b)
    c = _lin(_lin(cond_ref[...], ce0_W, ce0_b), ce1_W, ce1_b)
    gl = jnp.concatenate([gl, c], axis=1)
    gl = _lin(gl, fc0_W, fc0_b)
    gl = _lin(gl, fc1_W, fc1_b)
    gl = _lin(gl, fc2_W, fc2_b)
    t = gl * fc3_W[...]
    s = jnp.dot(t, jnp.ones((32, V), f32),
                preferred_element_type=f32)
    out_ref[...] = s + fc3_b[0, 0]


@functools.partial(jax.jit, static_argnames=())
def kernel(edges, hidden, nodes, node_params, cond,
           ne0_W, ne0_b, ne1_W, ne1_b, g1_W, g1_b, g2_W, g2_b,
           g3a_W, g3a_b, g3b_W, g3b_b, ce0_W, ce0_b, ce1_W, ce1_b,
           fc0_W, fc0_b, fc1_W, fc1_b, fc2_W, fc2_b, fc3_W, fc3_b):
    del hidden
    edges2d = edges.transpose(0, 1, 3, 2).reshape(B * V * C, V)
    nodes2d = nodes.reshape(B * V, -1)
    nparams2d = node_params.reshape(B * V, -1)
    args = [edges2d, nodes2d, nparams2d, cond,
            ne0_W, ne0_b.reshape(1, -1), ne1_W, ne1_b.reshape(1, -1),
            g1_W, g1_b.reshape(1, -1), g2_W, g2_b.reshape(1, -1),
            g3a_W, g3a_b.reshape(1, -1), g3b_W, g3b_b.reshape(1, -1),
            ce0_W, ce0_b.reshape(1, -1), ce1_W, ce1_b.reshape(1, -1),
            fc0_W, fc0_b.reshape(1, -1), fc1_W, fc1_b.reshape(1, -1),
            fc2_W, fc2_b.reshape(1, -1), fc3_W, fc3_b.reshape(1, -1)]
    out = pl.pallas_call(
        _fused_kernel,
        out_shape=jax.ShapeDtypeStruct((B, V), jnp.float32),
    )(*args)
    return out[:, :1]
